# baseline (speedup 1.0000x reference)
import functools

import jax
import jax.numpy as jnp
from jax import lax
from jax.experimental import pallas as pl
from jax.experimental.pallas import tpu as pltpu
from jax.experimental.pallas import tpu_sc as plsc

M = 8192
N = 16384
C = 128
BN = 1024
_SPATIAL = 128.0
_UNIT = 0.4


NCH = 64
W = M // NCH


def _top3_body(vx_ref, tT_ref, t2T_ref, q2c_ref, idx_ref, w_ref):
    ab = jnp.dot(vx_ref[...], tT_ref[...],
                 preferred_element_type=jnp.float32)
    d2 = jnp.maximum((t2T_ref[...] + q2c_ref[...]) - 2.0 * ab, 0.0)
    d3 = d2.reshape(NCH, W, BN)
    BIG = jnp.int32(M)
    INF = jnp.float32(jnp.inf)

    cm = jnp.min(d3, axis=1)
    ci = lax.broadcasted_iota(jnp.int32, cm.shape, 0)
    NB = jnp.int32(NCH)
    c1v = jnp.min(cm, axis=0, keepdims=True)
    c1 = jnp.min(jnp.where(cm == c1v, ci, NB), axis=0, keepdims=True)
    c2v = jnp.min(jnp.where(ci == c1, INF, cm), axis=0, keepdims=True)
    c2 = jnp.min(jnp.where((cm == c2v) & (ci != c1), ci, NB),
                 axis=0, keepdims=True)
    c3v = jnp.min(jnp.where((ci == c1) | (ci == c2), INF, cm),
                  axis=0, keepdims=True)
    c3 = jnp.min(jnp.where((cm == c3v) & (ci != c1) & (ci != c2), ci, NB),
                 axis=0, keepdims=True)

    ci3 = lax.broadcasted_iota(jnp.int32, (NCH, 1, BN), 0)
    g1 = jnp.min(jnp.where(ci3 == c1[None], d3, INF), axis=0)
    g2 = jnp.min(jnp.where(ci3 == c2[None], d3, INF), axis=0)
    g3 = jnp.min(jnp.where(ci3 == c3[None], d3, INF), axis=0)
    cand = jnp.concatenate([g1, g2, g3], axis=0)
    iw = lax.broadcasted_iota(jnp.int32, (W, BN), 0)
    gidx = jnp.concatenate([c1 * W + iw, c2 * W + iw, c3 * W + iw], axis=0)

    m1 = jnp.min(cand, axis=0, keepdims=True)
    i1 = jnp.min(jnp.where(cand == m1, gidx, BIG), axis=0, keepdims=True)
    m2 = jnp.min(jnp.where(gidx == i1, INF, cand), axis=0, keepdims=True)
    i2 = jnp.min(jnp.where((cand == m2) & (gidx != i1), gidx, BIG),
                 axis=0, keepdims=True)
    m3 = jnp.min(jnp.where((gidx == i1) | (gidx == i2), INF, cand),
                 axis=0, keepdims=True)
    i3 = jnp.min(jnp.where((cand == m3) & (gidx != i1) & (gidx != i2),
                           gidx, BIG), axis=0, keepdims=True)

    r1 = 1.0 / (m1 + 1e-8)
    r2 = 1.0 / (m2 + 1e-8)
    r3 = 1.0 / (m3 + 1e-8)
    s = r1 + r2 + r3
    zi = jnp.zeros_like(i1)
    zf = jnp.zeros_like(m1)
    idx_ref[...] = jnp.concatenate([i1, i2, i3, zi], axis=0)
    w_ref[...] = jnp.concatenate([r1 / s, r2 / s, r3 / s, zf], axis=0)


def _top3(targets, vxt2, t2T, q2c):
    grid = N // BN
    return pl.pallas_call(
        _top3_body,
        grid=(grid,),
        in_specs=[
            pl.BlockSpec((M, 4), lambda i: (0, 0)),
            pl.BlockSpec((4, BN), lambda i: (0, i)),
            pl.BlockSpec((1, BN), lambda i: (0, i)),
            pl.BlockSpec((M, 1), lambda i: (0, 0)),
        ],
        out_specs=[
            pl.BlockSpec((4, BN), lambda i: (0, i)),
            pl.BlockSpec((4, BN), lambda i: (0, i)),
        ],
        out_shape=[
            jax.ShapeDtypeStruct((4, N), jnp.int32),
            jax.ShapeDtypeStruct((4, N), jnp.float32),
        ],
    )(targets, vxt2, t2T, q2c)


_NC, _NS = 2, 16
_NW = _NC * _NS
_ROWS = 3 * N
_PC = 32
_RC = 3 * _PC
_PPW = N // _NW
_NCHK = _PPW // _PC
_L = 16


def _sc_interp(feats, idx_pm, w_pm):
    mesh = plsc.VectorSubcoreMesh(core_axis_name="c", subcore_axis_name="s")

    @functools.partial(
        pl.kernel,
        mesh=mesh,
        out_type=jax.ShapeDtypeStruct((N, C), jnp.float32),
        scratch_types=[
            pltpu.VMEM((_RC,), jnp.int32),
            pltpu.VMEM((_RC,), jnp.int32),
            pltpu.VMEM((_RC, _L), jnp.float32),
            pltpu.VMEM((_RC, _L), jnp.float32),
            pltpu.VMEM((_RC, C), jnp.float32),
            pltpu.VMEM((_RC, C), jnp.float32),
            pltpu.VMEM((_PC, C), jnp.float32),
            pltpu.VMEM((_PC, C), jnp.float32),
            pltpu.SemaphoreType.DMA,
            pltpu.SemaphoreType.DMA,
        ],
    )
    def interp_kernel(feats_hbm, idx_hbm, w_hbm, out_hbm,
                      idx0, idx1, w0, w1, rows0, rows1, out0, out1,
                      sem0, sem1):
        wid = lax.axis_index("s") * _NC + lax.axis_index("c")
        rbase = wid * (3 * _PPW)
        pbase = wid * _PPW

        def compute(rows_v, w_c, out_v):
            for p in range(_PC):
                wk = [w_c[3 * p + k] for k in range(3)]
                for cc in range(C // _L):
                    sl = pl.ds(cc * _L, _L)
                    out_v[p, sl] = (rows_v[3 * p, sl] * wk[0]
                                    + rows_v[3 * p + 1, sl] * wk[1]
                                    + rows_v[3 * p + 2, sl] * wk[2])

        pltpu.sync_copy(idx_hbm.at[pl.ds(rbase, _RC)], idx0)
        pltpu.sync_copy(w_hbm.at[pl.ds(rbase, _RC)], w0)
        pltpu.async_copy(feats_hbm.at[idx0], rows0, sem0)

        def body(g, carry):
            cA = 2 * g
            cB = cA + 1
            roffB = rbase + cB * _RC
            pltpu.sync_copy(idx_hbm.at[pl.ds(roffB, _RC)], idx1)
            pltpu.sync_copy(w_hbm.at[pl.ds(roffB, _RC)], w1)
            pltpu.async_copy(feats_hbm.at[idx1], rows1, sem1)
            pltpu.make_async_copy(feats_hbm.at[idx0], rows0, sem0).wait()
            compute(rows0, w0, out0)
            pltpu.sync_copy(out0, out_hbm.at[pl.ds(pbase + cA * _PC, _PC)])

            @pl.when(g < _NCHK // 2 - 1)
            def _():
                roffA = rbase + (cA + 2) * _RC
                pltpu.sync_copy(idx_hbm.at[pl.ds(roffA, _RC)], idx0)
                pltpu.sync_copy(w_hbm.at[pl.ds(roffA, _RC)], w0)
                pltpu.async_copy(feats_hbm.at[idx0], rows0, sem0)

            pltpu.make_async_copy(feats_hbm.at[idx1], rows1, sem1).wait()
            compute(rows1, w1, out1)
            pltpu.sync_copy(out1, out_hbm.at[pl.ds(pbase + cB * _PC, _PC)])
            return carry

        lax.fori_loop(0, _NCHK // 2, body, 0)

    return interp_kernel(feats, idx_pm, w_pm)


def kernel(sparse_features, sparse_indices, point_cloud, batch_ids):
    unit = jnp.full((3,), _UNIT, dtype=jnp.float32)
    voxel_extent = jnp.full((3,), _UNIT * _SPATIAL, dtype=jnp.float32)
    occ = sparse_indices.astype(jnp.float32)
    vx_xyz = occ[:, 1:] * unit - 0.5 * voxel_extent + 0.5 * unit
    vx_points = jnp.concatenate([occ[:, :1], vx_xyz], axis=1)
    targets = jnp.concatenate(
        [batch_ids.astype(jnp.float32)[:, None], point_cloud], axis=1)
    t2T = jnp.sum(targets * targets, axis=1)[None, :]
    q2c = jnp.sum(vx_points * vx_points, axis=1)[:, None]
    tT = targets.T

    idx4T, w4T = _top3(vx_points, tT, t2T, q2c)
    idx_pm = idx4T[:3].T.reshape(_ROWS)
    w_exp = jnp.broadcast_to(w4T[:3].T.reshape(_ROWS, 1), (_ROWS, _L))
    return _sc_interp(sparse_features, idx_pm, w_exp)


# speedup vs baseline: 1.1200x; 1.1200x over previous
import functools

import jax
import jax.numpy as jnp
from jax import lax
from jax.experimental import pallas as pl
from jax.experimental.pallas import tpu as pltpu
from jax.experimental.pallas import tpu_sc as plsc

M = 8192
N = 16384
C = 128
BN = 1024
_SPATIAL = 128.0
_UNIT = 0.4


NCH = 64
W = M // NCH


def _top3_body(vx_ref, tT_ref, t2T_ref, q2c_ref, idx_ref, w_ref):
    ab = jnp.dot(vx_ref[...], tT_ref[...],
                 preferred_element_type=jnp.float32)
    d2 = jnp.maximum((t2T_ref[...] + q2c_ref[...]) - 2.0 * ab, 0.0)
    d3 = d2.reshape(NCH, W, BN)
    BIG = jnp.int32(M)
    INF = jnp.float32(jnp.inf)

    cm = jnp.min(d3, axis=1)
    ci = lax.broadcasted_iota(jnp.int32, cm.shape, 0)
    NB = jnp.int32(NCH)
    c1v = jnp.min(cm, axis=0, keepdims=True)
    c1 = jnp.min(jnp.where(cm == c1v, ci, NB), axis=0, keepdims=True)
    c2v = jnp.min(jnp.where(ci == c1, INF, cm), axis=0, keepdims=True)
    c2 = jnp.min(jnp.where((cm == c2v) & (ci != c1), ci, NB),
                 axis=0, keepdims=True)
    c3v = jnp.min(jnp.where((ci == c1) | (ci == c2), INF, cm),
                  axis=0, keepdims=True)
    c3 = jnp.min(jnp.where((cm == c3v) & (ci != c1) & (ci != c2), ci, NB),
                 axis=0, keepdims=True)

    ci3 = lax.broadcasted_iota(jnp.int32, (NCH, 1, BN), 0)
    g1 = jnp.min(jnp.where(ci3 == c1[None], d3, INF), axis=0)
    g2 = jnp.min(jnp.where(ci3 == c2[None], d3, INF), axis=0)
    g3 = jnp.min(jnp.where(ci3 == c3[None], d3, INF), axis=0)
    cand = jnp.concatenate([g1, g2, g3], axis=0)
    iw = lax.broadcasted_iota(jnp.int32, (W, BN), 0)
    gidx = jnp.concatenate([c1 * W + iw, c2 * W + iw, c3 * W + iw], axis=0)

    m1 = jnp.min(cand, axis=0, keepdims=True)
    i1 = jnp.min(jnp.where(cand == m1, gidx, BIG), axis=0, keepdims=True)
    m2 = jnp.min(jnp.where(gidx == i1, INF, cand), axis=0, keepdims=True)
    i2 = jnp.min(jnp.where((cand == m2) & (gidx != i1), gidx, BIG),
                 axis=0, keepdims=True)
    m3 = jnp.min(jnp.where((gidx == i1) | (gidx == i2), INF, cand),
                 axis=0, keepdims=True)
    i3 = jnp.min(jnp.where((cand == m3) & (gidx != i1) & (gidx != i2),
                           gidx, BIG), axis=0, keepdims=True)

    r1 = 1.0 / (m1 + 1e-8)
    r2 = 1.0 / (m2 + 1e-8)
    r3 = 1.0 / (m3 + 1e-8)
    s = r1 + r2 + r3
    zi = jnp.zeros_like(i1)
    zf = jnp.zeros_like(m1)
    idx_ref[...] = jnp.concatenate([i1, i2, i3, zi], axis=0)
    w_ref[...] = jnp.concatenate([r1 / s, r2 / s, r3 / s, zf], axis=0)


def _top3(targets, vxt2, t2T, q2c):
    grid = N // BN
    return pl.pallas_call(
        _top3_body,
        grid=(grid,),
        in_specs=[
            pl.BlockSpec((M, 4), lambda i: (0, 0)),
            pl.BlockSpec((4, BN), lambda i: (0, i)),
            pl.BlockSpec((1, BN), lambda i: (0, i)),
            pl.BlockSpec((M, 1), lambda i: (0, 0)),
        ],
        out_specs=[
            pl.BlockSpec((4, BN), lambda i: (0, i)),
            pl.BlockSpec((4, BN), lambda i: (0, i)),
        ],
        out_shape=[
            jax.ShapeDtypeStruct((4, N), jnp.int32),
            jax.ShapeDtypeStruct((4, N), jnp.float32),
        ],
    )(targets, vxt2, t2T, q2c)


_NC, _NS = 2, 16
_NW = _NC * _NS
_ROWS = 3 * N
_RPW = _ROWS // _NW
_CH = 128


def _sc_gather(feats, idx_flat):
    mesh = plsc.VectorSubcoreMesh(core_axis_name="c", subcore_axis_name="s")

    @functools.partial(
        pl.kernel,
        mesh=mesh,
        out_type=jax.ShapeDtypeStruct((_ROWS, C), jnp.float32),
        scratch_types=[
            pltpu.VMEM((_CH,), jnp.int32),
            pltpu.VMEM((_CH, C), jnp.float32),
            pltpu.SemaphoreType.DMA,
        ],
    )
    def gather_kernel(feats_hbm, idx_hbm, out_hbm, idx_v, rows_v, sem):
        wid = lax.axis_index("s") * _NC + lax.axis_index("c")
        base = wid * _RPW

        def body(c, carry):
            off = base + c * _CH
            pltpu.sync_copy(idx_hbm.at[pl.ds(off, _CH)], idx_v)
            pltpu.async_copy(feats_hbm.at[idx_v], rows_v, sem).wait()
            pltpu.sync_copy(rows_v, out_hbm.at[pl.ds(off, _CH)])
            return carry

        lax.fori_loop(0, _RPW // _CH, body, 0)

    return gather_kernel(feats, idx_flat)


def _wsum_body(g_ref, w_ref, out_ref):
    g = g_ref[...]
    w = w_ref[...]
    out_ref[...] = (g[0] * w[:, 0:1]
                    + g[1] * w[:, 1:2]
                    + g[2] * w[:, 2:3])


def _wsum(gathered, w):
    grid = N // BN
    return pl.pallas_call(
        _wsum_body,
        grid=(grid,),
        in_specs=[
            pl.BlockSpec((3, BN, C), lambda i: (0, i, 0)),
            pl.BlockSpec((BN, 4), lambda i: (i, 0)),
        ],
        out_specs=pl.BlockSpec((BN, C), lambda i: (i, 0)),
        out_shape=jax.ShapeDtypeStruct((N, C), jnp.float32),
    )(gathered, w)


def kernel(sparse_features, sparse_indices, point_cloud, batch_ids):
    unit = jnp.full((3,), _UNIT, dtype=jnp.float32)
    voxel_extent = jnp.full((3,), _UNIT * _SPATIAL, dtype=jnp.float32)
    occ = sparse_indices.astype(jnp.float32)
    vx_xyz = occ[:, 1:] * unit - 0.5 * voxel_extent + 0.5 * unit
    vx_points = jnp.concatenate([occ[:, :1], vx_xyz], axis=1)
    targets = jnp.concatenate(
        [batch_ids.astype(jnp.float32)[:, None], point_cloud], axis=1)
    t2T = jnp.sum(targets * targets, axis=1)[None, :]
    q2c = jnp.sum(vx_points * vx_points, axis=1)[:, None]
    tT = targets.T

    idx4T, w4T = _top3(vx_points, tT, t2T, q2c)
    idx_flat = idx4T[:3].reshape(_ROWS)
    gathered = _sc_gather(sparse_features, idx_flat)
    return _wsum(gathered.reshape(3, N, C), w4T.T)
